# Initial kernel scaffold; baseline (speedup 1.0000x reference)
import jax
import jax.numpy as jnp
from jax.experimental import pallas as pl


def kernel(x, W):
    raise NotImplementedError("write your pallas kernel here")


# speedup vs baseline: 1.4858x; 1.4858x over previous
import jax
import jax.numpy as jnp
from jax.experimental import pallas as pl
from jax.experimental.pallas import tpu as pltpu
from jax.experimental.pallas import tpu_sc as plsc

N_TOKENS = 16 * 576
DIM = 256
K_CODES = 8192

TOKEN_BLOCK = 256
GATHER_WINDOW = 128


def _wsq_kernel(w_ref, o_ref):
    w = w_ref[...]
    o_ref[...] = jnp.sum(w * w, axis=1)[None, :]


def _argmin_kernel(x_ref, w_ref, wsq_ref, idx_ref):
    xb = x_ref[...]
    wb = w_ref[...]
    dot = jax.lax.dot_general(
        xb, wb, (((1,), (1,)), ((), ())),
        preferred_element_type=jnp.float32)
    x_sq = jnp.sum(xb * xb, axis=1, keepdims=True)
    s = (x_sq + wsq_ref[...]) - 2.0 * dot
    m = jnp.min(s, axis=1, keepdims=True)
    iota = jax.lax.broadcasted_iota(jnp.int32, s.shape, 1)
    idx = jnp.min(jnp.where(s == m, iota, K_CODES), axis=1)
    idx_ref[...] = idx[:, None]


def _compute_indices(xf, W):
    wsq = pl.pallas_call(
        _wsq_kernel,
        out_shape=jax.ShapeDtypeStruct((1, K_CODES), jnp.float32),
    )(W)
    idx = pl.pallas_call(
        _argmin_kernel,
        grid=(N_TOKENS // TOKEN_BLOCK,),
        in_specs=[
            pl.BlockSpec((TOKEN_BLOCK, DIM), lambda i: (i, 0)),
            pl.BlockSpec((K_CODES, DIM), lambda i: (0, 0)),
            pl.BlockSpec((1, K_CODES), lambda i: (0, 0)),
        ],
        out_specs=pl.BlockSpec((TOKEN_BLOCK, 1), lambda i: (i, 0)),
        out_shape=jax.ShapeDtypeStruct((N_TOKENS, 1), jnp.int32),
        compiler_params=pltpu.CompilerParams(
            dimension_semantics=("parallel",)),
    )(xf, W, wsq)
    return idx


def _sc_gather(W, idx_row):
    mesh = plsc.VectorSubcoreMesh(core_axis_name="core",
                                  subcore_axis_name="subcore")

    @pl.kernel(
        out_type=jax.ShapeDtypeStruct((N_TOKENS, DIM), jnp.float32),
        mesh=mesh)
    def kern(w_hbm, i_hbm, o_hbm):
        def body(i_vmem, o_vmem):
            pltpu.sync_copy(w_hbm.at[i_vmem.at[0]], o_vmem)

        pltpu.emit_pipeline(
            body,
            grid=(N_TOKENS // GATHER_WINDOW,),
            in_specs=[pl.BlockSpec((1, GATHER_WINDOW),
                                   index_map=lambda i: (0, i))],
            out_specs=[pl.BlockSpec((GATHER_WINDOW, DIM),
                                    index_map=lambda i: (i, 0))],
            core_axis_name=("core", "subcore"),
            dimension_semantics=(pltpu.PARALLEL,),
        )(i_hbm, o_hbm)

    return kern(W, idx_row)


def kernel(x, W):
    xf = x.reshape(-1, DIM)
    idx = _compute_indices(xf, W)
    idx_row = idx.reshape(1, N_TOKENS)
    quantized = _sc_gather(W, idx_row)
    return (quantized, idx_row)
